# baseline (speedup 1.0000x reference)
import functools

import jax
import jax.numpy as jnp
from jax.experimental import pallas as pl
from jax.experimental.pallas import tpu as pltpu

_M, _K, _N = 4096, 2048, 2048
_BM = 1024
_BN = 1024
_NI = _M // _BM
_NJ = _N // _BN


def _mm_kernel(expert_ref, x_hbm, w_hbm, o_hbm,
               x_vmem, w_buf, o_buf, x_sem, w_sem, o_sem):
    e = expert_ref[0]

    x_copies = []
    for i in range(_NI):
        c = pltpu.make_async_copy(
            x_hbm.at[pl.ds(i * _BM, _BM), :],
            x_vmem.at[pl.ds(i * _BM, _BM), :],
            x_sem.at[i])
        c.start()
        x_copies.append(c)

    w_copies = []
    for j in range(_NJ):
        c = pltpu.make_async_copy(
            w_hbm.at[e, pl.ds(j * _BN, _BN), :],
            w_buf.at[j],
            w_sem.at[j])
        c.start()
        w_copies.append(c)

    o_copies = [None, None]
    for j in range(_NJ):
        w_copies[j].wait()
        for i in range(_NI):
            if j == 0:
                x_copies[i].wait()
            ob = (j * _NI + i) % 2
            if o_copies[ob] is not None:
                o_copies[ob].wait()
            o_buf[ob] = jax.lax.dot_general(
                x_vmem[pl.ds(i * _BM, _BM), :], w_buf[j],
                dimension_numbers=(((1,), (1,)), ((), ())),
                preferred_element_type=jnp.float32,
            )
            c = pltpu.make_async_copy(
                o_buf.at[ob],
                o_hbm.at[pl.ds(i * _BM, _BM), pl.ds(j * _BN, _BN)],
                o_sem.at[ob])
            c.start()
            o_copies[ob] = c

    o_copies[0].wait()
    o_copies[1].wait()


@functools.partial(jax.jit, static_argnames=())
def kernel(state, expert_id, w):
    expert = jnp.asarray(expert_id, dtype=jnp.int32).reshape((1,))
    out = pl.pallas_call(
        _mm_kernel,
        grid_spec=pltpu.PrefetchScalarGridSpec(
            num_scalar_prefetch=1,
            grid=(1,),
            in_specs=[
                pl.BlockSpec(memory_space=pl.ANY),
                pl.BlockSpec(memory_space=pl.ANY),
            ],
            out_specs=pl.BlockSpec(memory_space=pl.ANY),
            scratch_shapes=[
                pltpu.VMEM((_M, _K), jnp.float32),
                pltpu.VMEM((_NJ, _BN, _K), jnp.float32),
                pltpu.VMEM((2, _BM, _BN), jnp.float32),
                pltpu.SemaphoreType.DMA((_NI,)),
                pltpu.SemaphoreType.DMA((_NJ,)),
                pltpu.SemaphoreType.DMA((2,)),
            ],
        ),
        out_shape=jax.ShapeDtypeStruct((_M, _N), jnp.float32),
    )(expert, state, w)
    return out


# speedup vs baseline: 1.0423x; 1.0423x over previous
import functools

import jax
import jax.numpy as jnp
from jax.experimental import pallas as pl
from jax.experimental.pallas import tpu as pltpu


def _matmul_kernel(expert_ref, x_ref, w_ref, o_ref):
    o_ref[...] = jax.lax.dot_general(
        x_ref[...], w_ref[0],
        dimension_numbers=(((1,), (1,)), ((), ())),
        preferred_element_type=jnp.float32,
    )


@functools.partial(jax.jit, static_argnames=())
def kernel(state, expert_id, w):
    M, K = state.shape
    E, N, K2 = w.shape
    BM, BN = 2048, 256
    expert = jnp.asarray(expert_id, dtype=jnp.int32).reshape((1,))

    grid = (M // BM, N // BN)
    out = pl.pallas_call(
        _matmul_kernel,
        grid_spec=pltpu.PrefetchScalarGridSpec(
            num_scalar_prefetch=1,
            grid=grid,
            in_specs=[
                pl.BlockSpec((BM, K), lambda i, j, e: (i, 0)),
                pl.BlockSpec((1, BN, K), lambda i, j, e: (e[0], j, 0)),
            ],
            out_specs=pl.BlockSpec((BM, BN), lambda i, j, e: (i, j)),
        ),
        out_shape=jax.ShapeDtypeStruct((M, N), jnp.float32),
        compiler_params=pltpu.CompilerParams(
            dimension_semantics=("arbitrary", "arbitrary"),
        ),
    )(expert, state, w)
    return out


# speedup vs baseline: 1.0878x; 1.0436x over previous
import functools

import jax
import jax.numpy as jnp
from jax.experimental import pallas as pl
from jax.experimental.pallas import tpu as pltpu


def _matmul_kernel(expert_ref, x_ref, w_ref, o_ref):
    o_ref[...] = jax.lax.dot_general(
        x_ref[...], w_ref[0],
        dimension_numbers=(((1,), (1,)), ((), ())),
        preferred_element_type=jnp.float32,
    )


@functools.partial(jax.jit, static_argnames=())
def kernel(state, expert_id, w):
    M, K = state.shape
    E, N, K2 = w.shape
    BM, BN = 2048, 512
    expert = jnp.asarray(expert_id, dtype=jnp.int32).reshape((1,))

    grid = (M // BM, N // BN)
    out = pl.pallas_call(
        _matmul_kernel,
        grid_spec=pltpu.PrefetchScalarGridSpec(
            num_scalar_prefetch=1,
            grid=grid,
            in_specs=[
                pl.BlockSpec((BM, K), lambda i, j, e: (i, 0)),
                pl.BlockSpec((1, BN, K), lambda i, j, e: (e[0], j, 0)),
            ],
            out_specs=pl.BlockSpec((BM, BN), lambda i, j, e: (i, j)),
        ),
        out_shape=jax.ShapeDtypeStruct((M, N), jnp.float32),
        compiler_params=pltpu.CompilerParams(
            dimension_semantics=("arbitrary", "arbitrary"),
        ),
    )(expert, state, w)
    return out
